# Initial kernel scaffold; baseline (speedup 1.0000x reference)
import jax
import jax.numpy as jnp
from jax.experimental import pallas as pl


def kernel(output, targets, anchors):
    raise NotImplementedError("write your pallas kernel here")


# speedup vs baseline: 5.8542x; 5.8542x over previous
import jax
import jax.numpy as jnp
from jax.experimental import pallas as pl
from jax.experimental.pallas import tpu as pltpu

_OBJECT_SCALE = 5.0
_NOOBJECT_SCALE = 1.0
_IGNORE_THRES = 0.6


def _slog(s):
    return jnp.max(jnp.log(jnp.broadcast_to(s, (8, 128))))


def _region_body(nB, nA, nH, nW):
    def body(out_ref, tgt_ref, anc_ref, loss_ref, r50_ref, r75_ref, aiou_ref, acc_ref):
        b = pl.program_id(0)

        t0 = tgt_ref[b, 0]
        t1 = tgt_ref[b, 1]
        t2 = tgt_ref[b, 2]
        t3 = tgt_ref[b, 3]
        gt_x = t0 * nW
        gt_y = t1 * nH
        gt_w = t2 * nW
        gt_h = t3 * nH
        gxf = jnp.floor(gt_x)
        gyf = jnp.floor(gt_y)
        gx = gxf.astype(jnp.int32)
        gy = gyf.astype(jnp.int32)

        aw = [anc_ref[a, 0] for a in range(nA)]
        ah = [anc_ref[a, 1] for a in range(nA)]

        ratios = []
        for a in range(nA):
            inter = jnp.minimum(gt_w, aw[a]) * jnp.minimum(gt_h, ah[a])
            union = gt_w * gt_h + 1e-16 + aw[a] * ah[a] - inter
            ratios.append(inter / union)
        best = ratios[0]
        for a in range(1, nA):
            best = jnp.maximum(best, ratios[a])
        sels = []
        found = ratios[0] < ratios[0]
        for a in range(nA):
            is_best = jnp.logical_and(ratios[a] >= best, jnp.logical_not(found))
            sels.append(is_best)
            found = jnp.logical_or(found, is_best)

        b2x1 = gt_x - gt_w / 2
        b2x2 = gt_x + gt_w / 2
        b2y1 = gt_y - gt_h / 2
        b2y2 = gt_y + gt_h / 2

        row = jax.lax.broadcasted_iota(jnp.int32, (nH, nW), 0)
        col = jax.lax.broadcasted_iota(jnp.int32, (nH, nW), 1)
        rowf = row.astype(jnp.float32)
        colf = col.astype(jnp.float32)
        cell_eq = jnp.logical_and(row == gy, col == gx)

        s_n2 = 0.0
        s_cnt = 0.0
        g_x = 0.0
        g_y = 0.0
        g_w = 0.0
        g_h = 0.0
        g_conf = 0.0
        g_pw = 0.0
        g_ph = 0.0

        for a in range(nA):
            base = 5 * a
            xo = out_ref[0, base + 0]
            yo = out_ref[0, base + 1]
            wo = out_ref[0, base + 2]
            ho = out_ref[0, base + 3]
            co = out_ref[0, base + 4]
            x = 1.0 / (1.0 + jnp.exp(-xo))
            y = 1.0 / (1.0 + jnp.exp(-yo))
            conf = 1.0 / (1.0 + jnp.exp(-co))
            px = x + colf
            py = y + rowf
            pw = jnp.exp(wo) * aw[a]
            ph = jnp.exp(ho) * ah[a]

            b1x1 = px - pw / 2
            b1x2 = px + pw / 2
            b1y1 = py - ph / 2
            b1y2 = py + ph / 2
            ix1 = jnp.maximum(b1x1, b2x1)
            iy1 = jnp.maximum(b1y1, b2y1)
            ix2 = jnp.minimum(b1x2, b2x2)
            iy2 = jnp.minimum(b1y2, b2y2)
            inter = jnp.maximum(ix2 - ix1 + 1.0, 0.0) * jnp.maximum(iy2 - iy1 + 1.0, 0.0)
            a1 = (b1x2 - b1x1 + 1.0) * (b1y2 - b1y1 + 1.0)
            a2 = (b2x2 - b2x1 + 1.0) * (b2y2 - b2y1 + 1.0)
            iou = inter / (a1 + a2 - inter + 1e-16)

            mask_obj = jnp.logical_and(cell_eq, sels[a])
            noobj = jnp.logical_and(jnp.logical_not(mask_obj), iou <= _IGNORE_THRES)
            cm = jnp.where(noobj, conf, 0.0)
            s_n2 = s_n2 + jnp.sum(cm * cm)
            s_cnt = s_cnt + jnp.sum(noobj.astype(jnp.float32))

            g_x = g_x + jnp.sum(jnp.where(mask_obj, x, 0.0))
            g_y = g_y + jnp.sum(jnp.where(mask_obj, y, 0.0))
            g_w = g_w + jnp.sum(jnp.where(mask_obj, wo, 0.0))
            g_h = g_h + jnp.sum(jnp.where(mask_obj, ho, 0.0))
            g_conf = g_conf + jnp.sum(jnp.where(mask_obj, conf, 0.0))
            g_pw = g_pw + jnp.sum(jnp.where(mask_obj, pw, 0.0))
            g_ph = g_ph + jnp.sum(jnp.where(mask_obj, ph, 0.0))

        a_w_best = 0.0
        a_h_best = 0.0
        for a in range(nA):
            a_w_best = a_w_best + jnp.where(sels[a], aw[a], 0.0)
            a_h_best = a_h_best + jnp.where(sels[a], ah[a], 0.0)

        tx = gt_x - gxf
        ty = gt_y - gyf
        tw = _slog(gt_w / a_w_best + 1e-16)
        th = _slog(gt_h / a_h_best + 1e-16)
        scale = 2.0 - t2 * t3

        sq_x = (g_x * scale - tx * scale) ** 2
        sq_y = (g_y * scale - ty * scale) ** 2
        sq_w = (g_w * scale - tw * scale) ** 2
        sq_h = (g_h * scale - th * scale) ** 2
        sq_conf = (g_conf - 1.0) ** 2

        px_o = g_x + gxf
        py_o = g_y + gyf
        p1x1 = px_o - g_pw / 2
        p1x2 = px_o + g_pw / 2
        p1y1 = py_o - g_ph / 2
        p1y2 = py_o + g_ph / 2
        jx1 = jnp.maximum(p1x1, b2x1)
        jy1 = jnp.maximum(p1y1, b2y1)
        jx2 = jnp.minimum(p1x2, b2x2)
        jy2 = jnp.minimum(p1y2, b2y2)
        jinter = jnp.maximum(jx2 - jx1 + 1.0, 0.0) * jnp.maximum(jy2 - jy1 + 1.0, 0.0)
        ja1 = (p1x2 - p1x1 + 1.0) * (p1y2 - p1y1 + 1.0)
        ja2 = (b2x2 - b2x1 + 1.0) * (b2y2 - b2y1 + 1.0)
        iou_v = jinter / (ja1 + ja2 - jinter + 1e-16)

        @pl.when(b == 0)
        def _init():
            for i in range(10):
                acc_ref[i] = 0.0

        vals = [
            sq_x, sq_y, sq_w, sq_h, sq_conf, s_n2, s_cnt,
            jnp.where(iou_v > 0.5, 1.0, 0.0),
            jnp.where(iou_v > 0.75, 1.0, 0.0),
            iou_v,
        ]
        for i, v in enumerate(vals):
            acc_ref[i] = acc_ref[i] + v

        @pl.when(b == nB - 1)
        def _fin():
            fnB = float(nB)
            n_noobj = jnp.maximum(acc_ref[6], 1.0)
            loss = (acc_ref[0] + acc_ref[1] + acc_ref[2] + acc_ref[3]
                    + _OBJECT_SCALE * acc_ref[4]) / fnB \
                + _NOOBJECT_SCALE * acc_ref[5] / n_noobj
            loss_ref[0] = loss
            r50_ref[0] = acc_ref[7] / fnB
            r75_ref[0] = acc_ref[8] / fnB
            aiou_ref[0] = acc_ref[9] / fnB

    return body


def kernel(output, targets, anchors):
    nB, C, nH, nW = output.shape
    nA = anchors.shape[0]
    body = _region_body(nB, nA, nH, nW)
    outs = pl.pallas_call(
        body,
        grid=(nB,),
        in_specs=[
            pl.BlockSpec((1, C, nH, nW), lambda b: (b, 0, 0, 0)),
            pl.BlockSpec(memory_space=pltpu.SMEM),
            pl.BlockSpec(memory_space=pltpu.SMEM),
        ],
        out_specs=[pl.BlockSpec(memory_space=pltpu.SMEM)] * 4,
        out_shape=[jax.ShapeDtypeStruct((1,), jnp.float32)] * 4,
        scratch_shapes=[pltpu.SMEM((10,), jnp.float32)],
    )(output, targets, anchors)
    return tuple(o[0] for o in outs)
